# baseline (speedup 1.0000x reference)
import jax
import jax.numpy as jnp
from jax import lax
from jax.experimental import pallas as pl
from jax.experimental.pallas import tpu as pltpu
from jax.experimental.pallas import tpu_sc as plsc

N = 100000
D = 128
S = 256
CH = 128
NFULL = N // CH
REM = N - NFULL * CH
NC = 2
NSUB = 16
NW = NC * NSUB
NK = NFULL // NW
EXTRA = NFULL - NK * NW
SEGS_PER_TILE = S // NSUB
NBUF = 6
DEFER = 3


def _sc_body(feat_hbm, ids_hbm, psum_hbm, pcnt_hbm,
             fbuf0, fbuf1, fbuf2, fbuf3, fbuf4, fbuf5,
             ibuf0, ibuf1, ibuf2, ibuf3, ibuf4, ibuf5,
             ones, obuf, cbuf, rem_f, rem_i,
             acc, cnt, lsem0, lsem1, lsem2, lsem3, lsem4, lsem5,
             ssem0, ssem1, ssem2, ssem3, ssem4, ssem5):
    cid = lax.axis_index("c")
    sid = lax.axis_index("s")
    wid = sid * NC + cid
    fbufs = (fbuf0, fbuf1, fbuf2, fbuf3, fbuf4, fbuf5)
    ibufs = (ibuf0, ibuf1, ibuf2, ibuf3, ibuf4, ibuf5)
    lsems = (lsem0, lsem1, lsem2, lsem3, lsem4, lsem5)
    ssems = (ssem0, ssem1, ssem2, ssem3, ssem4, ssem5)

    one = jnp.ones((16,), jnp.float32)
    z = jnp.zeros((16,), jnp.float32)
    for r in range(CH):
        for c in range(D // 16):
            ones[r, pl.ds(c * 16, 16)] = one
    for r in range(SEGS_PER_TILE):
        for c in range(D // 16):
            obuf[r, pl.ds(c * 16, 16)] = z
            cbuf[r, pl.ds(c * 16, 16)] = z

    seg0 = sid * SEGS_PER_TILE
    pltpu.sync_copy(obuf, acc.at[pl.ds(seg0, SEGS_PER_TILE)])
    pltpu.sync_copy(cbuf, cnt.at[pl.ds(seg0, SEGS_PER_TILE)])
    plsc.subcore_barrier()

    def start_load(k, b):
        start = (wid + k * NW) * CH
        pltpu.async_copy(ids_hbm.at[pl.ds(start, CH)], ibufs[b], lsems[b])
        pltpu.async_copy(feat_hbm.at[pl.ds(start, CH)], fbufs[b], lsems[b])

    def wait_load(b):
        pltpu.make_async_copy(ids_hbm.at[pl.ds(0, CH)], ibufs[b], lsems[b]).wait()
        pltpu.make_async_copy(feat_hbm.at[pl.ds(0, CH)], fbufs[b], lsems[b]).wait()

    def wait_scatter(b):
        pltpu.make_async_copy(fbufs[b], acc.at[ibufs[b]], ssems[b]).wait()
        pltpu.make_async_copy(ones, cnt.at[ibufs[b]], ssems[b]).wait()

    for b in range(NBUF):
        start_load(b, b)

    def pipe_body(k4, carry):
        for b in range(NBUF):
            k = k4 * NBUF + b
            wait_load(b)
            pltpu.async_copy(fbufs[b], acc.at[ibufs[b]], ssems[b], add=True)
            pltpu.async_copy(ones, cnt.at[ibufs[b]], ssems[b], add=True)

            @pl.when(k >= DEFER)
            def _():
                b2 = (b + DEFER) % NBUF
                wait_scatter(b2)

                @pl.when(k + DEFER < NK)
                def _():
                    start_load(k + DEFER, (b + DEFER) % NBUF)
        return carry

    lax.fori_loop(0, NK // NBUF, pipe_body, 0)
    for tail in range(DEFER):
        wait_scatter((NK - DEFER + tail) % NBUF)

    @pl.when(wid < EXTRA)
    def _():
        start = (wid + NK * NW) * CH
        pltpu.sync_copy(ids_hbm.at[pl.ds(start, CH)], ibuf0)
        pltpu.sync_copy(feat_hbm.at[pl.ds(start, CH)], fbuf0)
        pltpu.sync_copy(fbuf0, acc.at[ibuf0], add=True)
        pltpu.sync_copy(ones, cnt.at[ibuf0], add=True)

    @pl.when(wid == NW - 1)
    def _():
        start = NFULL * CH
        pltpu.sync_copy(ids_hbm.at[pl.ds(start, REM)], rem_i)
        pltpu.sync_copy(feat_hbm.at[pl.ds(start, REM)], rem_f)
        pltpu.sync_copy(rem_f, acc.at[rem_i], add=True)
        pltpu.sync_copy(ones.at[pl.ds(0, REM)], cnt.at[rem_i], add=True)

    plsc.subcore_barrier()

    pltpu.sync_copy(acc.at[pl.ds(seg0, SEGS_PER_TILE)], obuf)
    pltpu.sync_copy(cnt.at[pl.ds(seg0, SEGS_PER_TILE)], cbuf)
    pltpu.sync_copy(obuf, psum_hbm.at[cid, pl.ds(seg0, SEGS_PER_TILE)])
    pltpu.sync_copy(cbuf, pcnt_hbm.at[cid, pl.ds(seg0, SEGS_PER_TILE)])


def _combine_body(ps_ref, pc_ref, o_ref):
    s = ps_ref[0] + ps_ref[1]
    c = jnp.maximum(pc_ref[0] + pc_ref[1], 1.0)
    o_ref[...] = s / c


@jax.jit
def _segment_mean(feat, ids32):
    mesh = plsc.VectorSubcoreMesh(
        core_axis_name="c", subcore_axis_name="s", num_cores=NC)
    f = pl.kernel(
        _sc_body,
        out_type=(
            jax.ShapeDtypeStruct((NC, S, D), jnp.float32),
            jax.ShapeDtypeStruct((NC, S, D), jnp.float32),
        ),
        mesh=mesh,
        scratch_types=[
            pltpu.VMEM((CH, D), jnp.float32),
            pltpu.VMEM((CH, D), jnp.float32),
            pltpu.VMEM((CH, D), jnp.float32),
            pltpu.VMEM((CH, D), jnp.float32),
            pltpu.VMEM((CH, D), jnp.float32),
            pltpu.VMEM((CH, D), jnp.float32),
            pltpu.VMEM((CH,), jnp.int32),
            pltpu.VMEM((CH,), jnp.int32),
            pltpu.VMEM((CH,), jnp.int32),
            pltpu.VMEM((CH,), jnp.int32),
            pltpu.VMEM((CH,), jnp.int32),
            pltpu.VMEM((CH,), jnp.int32),
            pltpu.VMEM((CH, D), jnp.float32),
            pltpu.VMEM((SEGS_PER_TILE, D), jnp.float32),
            pltpu.VMEM((SEGS_PER_TILE, D), jnp.float32),
            pltpu.VMEM((REM, D), jnp.float32),
            pltpu.VMEM((REM,), jnp.int32),
            pltpu.VMEM_SHARED((S, D), jnp.float32),
            pltpu.VMEM_SHARED((S, D), jnp.float32),
            pltpu.SemaphoreType.DMA,
            pltpu.SemaphoreType.DMA,
            pltpu.SemaphoreType.DMA,
            pltpu.SemaphoreType.DMA,
            pltpu.SemaphoreType.DMA,
            pltpu.SemaphoreType.DMA,
            pltpu.SemaphoreType.DMA,
            pltpu.SemaphoreType.DMA,
            pltpu.SemaphoreType.DMA,
            pltpu.SemaphoreType.DMA,
            pltpu.SemaphoreType.DMA,
            pltpu.SemaphoreType.DMA,
        ],
    )
    psum, pcnt = f(feat, ids32)
    combine = pl.pallas_call(
        _combine_body,
        out_shape=jax.ShapeDtypeStruct((S, D), jnp.float32),
    )
    return combine(psum, pcnt)


def kernel(feat, segment_ids):
    return _segment_mean(feat, segment_ids.astype(jnp.int32))


# speedup vs baseline: 1.0394x; 1.0394x over previous
import jax
import jax.numpy as jnp
from jax import lax
from jax.experimental import pallas as pl
from jax.experimental.pallas import tpu as pltpu
from jax.experimental.pallas import tpu_sc as plsc

N = 100000
D = 128
S = 256
CH = 128
NFULL = N // CH
REM = N - NFULL * CH
NC = 2
NSUB = 16
NW = NC * NSUB
NK = NFULL // NW
EXTRA = NFULL - NK * NW
SEGS_PER_TILE = S // NSUB
NBUF = 4
DEFER = 2


def _sc_body(feat_hbm, ids_hbm, psum_hbm, pcnt_hbm,
             fbuf0, fbuf1, fbuf2, fbuf3, ibuf0, ibuf1, ibuf2, ibuf3,
             ones, obuf, cbuf, rem_f, rem_i,
             acc, cnt, lsem0, lsem1, lsem2, lsem3,
             ssem0, ssem1, ssem2, ssem3):
    cid = lax.axis_index("c")
    sid = lax.axis_index("s")
    wid = sid * NC + cid
    fbufs = (fbuf0, fbuf1, fbuf2, fbuf3)
    ibufs = (ibuf0, ibuf1, ibuf2, ibuf3)
    lsems = (lsem0, lsem1, lsem2, lsem3)
    ssems = (ssem0, ssem1, ssem2, ssem3)

    one = jnp.ones((16,), jnp.float32)
    z = jnp.zeros((16,), jnp.float32)
    for r in range(CH):
        for c in range(D // 16):
            ones[r, pl.ds(c * 16, 16)] = one
    for r in range(SEGS_PER_TILE):
        for c in range(D // 16):
            obuf[r, pl.ds(c * 16, 16)] = z
            cbuf[r, pl.ds(c * 16, 16)] = z

    seg0 = sid * SEGS_PER_TILE
    pltpu.sync_copy(obuf, acc.at[pl.ds(seg0, SEGS_PER_TILE)])
    pltpu.sync_copy(cbuf, cnt.at[pl.ds(seg0, SEGS_PER_TILE)])
    plsc.subcore_barrier()

    def start_load(k, b):
        start = (wid + k * NW) * CH
        pltpu.async_copy(ids_hbm.at[pl.ds(start, CH)], ibufs[b], lsems[b])
        pltpu.async_copy(feat_hbm.at[pl.ds(start, CH)], fbufs[b], lsems[b])

    def wait_load(b):
        pltpu.make_async_copy(ids_hbm.at[pl.ds(0, CH)], ibufs[b], lsems[b]).wait()
        pltpu.make_async_copy(feat_hbm.at[pl.ds(0, CH)], fbufs[b], lsems[b]).wait()

    def wait_scatter(b):
        pltpu.make_async_copy(fbufs[b], acc.at[ibufs[b]], ssems[b]).wait()
        pltpu.make_async_copy(ones, cnt.at[ibufs[b]], ssems[b]).wait()

    for b in range(NBUF):
        start_load(b, b)

    def pipe_body(k4, carry):
        for b in range(NBUF):
            k = k4 * NBUF + b
            wait_load(b)
            pltpu.async_copy(fbufs[b], acc.at[ibufs[b]], ssems[b], add=True)
            pltpu.async_copy(ones, cnt.at[ibufs[b]], ssems[b], add=True)

            @pl.when(k >= DEFER)
            def _():
                b2 = (b + DEFER) % NBUF
                wait_scatter(b2)

                @pl.when(k + DEFER < NK)
                def _():
                    start_load(k + DEFER, (b + DEFER) % NBUF)
        return carry

    lax.fori_loop(0, NK // NBUF, pipe_body, 0)
    for tail in range(DEFER):
        wait_scatter((NK - DEFER + tail) % NBUF)

    @pl.when(wid < EXTRA)
    def _():
        start = (wid + NK * NW) * CH
        pltpu.sync_copy(ids_hbm.at[pl.ds(start, CH)], ibuf0)
        pltpu.sync_copy(feat_hbm.at[pl.ds(start, CH)], fbuf0)
        pltpu.sync_copy(fbuf0, acc.at[ibuf0], add=True)
        pltpu.sync_copy(ones, cnt.at[ibuf0], add=True)

    @pl.when(wid == NW - 1)
    def _():
        start = NFULL * CH
        pltpu.sync_copy(ids_hbm.at[pl.ds(start, REM)], rem_i)
        pltpu.sync_copy(feat_hbm.at[pl.ds(start, REM)], rem_f)
        pltpu.sync_copy(rem_f, acc.at[rem_i], add=True)
        pltpu.sync_copy(ones.at[pl.ds(0, REM)], cnt.at[rem_i], add=True)

    plsc.subcore_barrier()

    pltpu.sync_copy(acc.at[pl.ds(seg0, SEGS_PER_TILE)], obuf)
    pltpu.sync_copy(cnt.at[pl.ds(seg0, SEGS_PER_TILE)], cbuf)
    pltpu.sync_copy(obuf, psum_hbm.at[cid, pl.ds(seg0, SEGS_PER_TILE)])
    pltpu.sync_copy(cbuf, pcnt_hbm.at[cid, pl.ds(seg0, SEGS_PER_TILE)])


def _combine_body(ps_ref, pc_ref, o_ref):
    s = ps_ref[0] + ps_ref[1]
    c = jnp.maximum(pc_ref[0] + pc_ref[1], 1.0)
    o_ref[...] = s / c


@jax.jit
def _segment_mean(feat, ids32):
    mesh = plsc.VectorSubcoreMesh(
        core_axis_name="c", subcore_axis_name="s", num_cores=NC)
    f = pl.kernel(
        _sc_body,
        out_type=(
            jax.ShapeDtypeStruct((NC, S, D), jnp.float32),
            jax.ShapeDtypeStruct((NC, S, D), jnp.float32),
        ),
        mesh=mesh,
        scratch_types=[
            pltpu.VMEM((CH, D), jnp.float32),
            pltpu.VMEM((CH, D), jnp.float32),
            pltpu.VMEM((CH, D), jnp.float32),
            pltpu.VMEM((CH, D), jnp.float32),
            pltpu.VMEM((CH,), jnp.int32),
            pltpu.VMEM((CH,), jnp.int32),
            pltpu.VMEM((CH,), jnp.int32),
            pltpu.VMEM((CH,), jnp.int32),
            pltpu.VMEM((CH, D), jnp.float32),
            pltpu.VMEM((SEGS_PER_TILE, D), jnp.float32),
            pltpu.VMEM((SEGS_PER_TILE, D), jnp.float32),
            pltpu.VMEM((REM, D), jnp.float32),
            pltpu.VMEM((REM,), jnp.int32),
            pltpu.VMEM_SHARED((S, D), jnp.float32),
            pltpu.VMEM_SHARED((S, D), jnp.float32),
            pltpu.SemaphoreType.DMA,
            pltpu.SemaphoreType.DMA,
            pltpu.SemaphoreType.DMA,
            pltpu.SemaphoreType.DMA,
            pltpu.SemaphoreType.DMA,
            pltpu.SemaphoreType.DMA,
            pltpu.SemaphoreType.DMA,
            pltpu.SemaphoreType.DMA,
        ],
    )
    psum, pcnt = f(feat, ids32)
    combine = pl.pallas_call(
        _combine_body,
        out_shape=jax.ShapeDtypeStruct((S, D), jnp.float32),
    )
    return combine(psum, pcnt)


def kernel(feat, segment_ids):
    return _segment_mean(feat, segment_ids.astype(jnp.int32))


# speedup vs baseline: 1.0527x; 1.0129x over previous
import jax
import jax.numpy as jnp
from jax import lax
from jax.experimental import pallas as pl
from jax.experimental.pallas import tpu as pltpu
from jax.experimental.pallas import tpu_sc as plsc

N = 100000
D = 128
S = 256
CH = 128
NFULL = N // CH
REM = N - NFULL * CH
NC = 2
NSUB = 16
NW = NC * NSUB
NK = NFULL // NW
EXTRA = NFULL - NK * NW
SEGS_PER_TILE = S // NSUB
NBUF = 4
DEFER = 2


def _sc_body(feat_hbm, ids_hbm, psum_hbm, pcnt_hbm,
             fbuf0, fbuf1, fbuf2, fbuf3, ibuf0, ibuf1, ibuf2, ibuf3,
             ones, obuf, cbuf, rem_f, rem_i,
             acc, cnt, lsem0, lsem1, lsem2, lsem3,
             ssem0, ssem1, ssem2, ssem3):
    cid = lax.axis_index("c")
    sid = lax.axis_index("s")
    wid = sid * NC + cid
    fbufs = (fbuf0, fbuf1, fbuf2, fbuf3)
    ibufs = (ibuf0, ibuf1, ibuf2, ibuf3)
    lsems = (lsem0, lsem1, lsem2, lsem3)
    ssems = (ssem0, ssem1, ssem2, ssem3)

    one = jnp.ones((16,), jnp.float32)
    z = jnp.zeros((16,), jnp.float32)
    for r in range(CH):
        for c in range(D // 16):
            ones[r, pl.ds(c * 16, 16)] = one
    for r in range(SEGS_PER_TILE):
        for c in range(D // 16):
            obuf[r, pl.ds(c * 16, 16)] = z
            cbuf[r, pl.ds(c * 16, 16)] = z

    seg0 = sid * SEGS_PER_TILE
    pltpu.sync_copy(obuf, acc.at[pl.ds(seg0, SEGS_PER_TILE)])
    pltpu.sync_copy(cbuf, cnt.at[pl.ds(seg0, SEGS_PER_TILE)])
    plsc.subcore_barrier()

    def start_load(k, b):
        start = (wid + k * NW) * CH
        pltpu.async_copy(ids_hbm.at[pl.ds(start, CH)], ibufs[b], lsems[b])
        pltpu.async_copy(feat_hbm.at[pl.ds(start, CH)], fbufs[b], lsems[b])

    def wait_load(b):
        pltpu.make_async_copy(ids_hbm.at[pl.ds(0, CH)], ibufs[b], lsems[b]).wait()
        pltpu.make_async_copy(feat_hbm.at[pl.ds(0, CH)], fbufs[b], lsems[b]).wait()

    def wait_scatter(b):
        pltpu.make_async_copy(fbufs[b], acc.at[ibufs[b]], ssems[b]).wait()
        pltpu.make_async_copy(ones, cnt.at[ibufs[b]], ssems[b]).wait()

    for b in range(NBUF):
        start_load(b, b)

    def pipe_body(k4, carry):
        for b in range(NBUF):
            k = k4 * NBUF + b
            wait_load(b)
            pltpu.async_copy(fbufs[b], acc.at[ibufs[b]], ssems[b], add=True)
            pltpu.async_copy(ones, cnt.at[ibufs[b]], ssems[b], add=True)

            @pl.when(k >= DEFER)
            def _():
                b2 = (b + DEFER) % NBUF
                wait_scatter(b2)

                @pl.when(k + DEFER < NK)
                def _():
                    start_load(k + DEFER, (b + DEFER) % NBUF)
        return carry

    lax.fori_loop(0, NK // NBUF, pipe_body, 0)

    @pl.when(wid < EXTRA)
    def _():
        start_load(NK, 0)

    for tail in range(DEFER):
        wait_scatter((NK - DEFER + tail) % NBUF)

    @pl.when(wid < EXTRA)
    def _():
        wait_load(0)
        pltpu.async_copy(fbuf0, acc.at[ibuf0], ssem0, add=True)
        pltpu.async_copy(ones, cnt.at[ibuf0], ssem0, add=True)
        wait_scatter(0)

    @pl.when(wid == NW - 1)
    def _():
        start = NFULL * CH
        pltpu.sync_copy(ids_hbm.at[pl.ds(start, REM)], rem_i)
        pltpu.sync_copy(feat_hbm.at[pl.ds(start, REM)], rem_f)
        pltpu.sync_copy(rem_f, acc.at[rem_i], add=True)
        pltpu.sync_copy(ones.at[pl.ds(0, REM)], cnt.at[rem_i], add=True)

    plsc.subcore_barrier()

    pltpu.sync_copy(acc.at[pl.ds(seg0, SEGS_PER_TILE)],
                    psum_hbm.at[cid, pl.ds(seg0, SEGS_PER_TILE)])
    pltpu.sync_copy(cnt.at[pl.ds(seg0, SEGS_PER_TILE)],
                    pcnt_hbm.at[cid, pl.ds(seg0, SEGS_PER_TILE)])


def _combine_body(ps_ref, pc_ref, o_ref):
    s = ps_ref[0] + ps_ref[1]
    c = jnp.maximum(pc_ref[0] + pc_ref[1], 1.0)
    o_ref[...] = s / c


@jax.jit
def _segment_mean(feat, ids32):
    mesh = plsc.VectorSubcoreMesh(
        core_axis_name="c", subcore_axis_name="s", num_cores=NC)
    f = pl.kernel(
        _sc_body,
        out_type=(
            jax.ShapeDtypeStruct((NC, S, D), jnp.float32),
            jax.ShapeDtypeStruct((NC, S, D), jnp.float32),
        ),
        mesh=mesh,
        scratch_types=[
            pltpu.VMEM((CH, D), jnp.float32),
            pltpu.VMEM((CH, D), jnp.float32),
            pltpu.VMEM((CH, D), jnp.float32),
            pltpu.VMEM((CH, D), jnp.float32),
            pltpu.VMEM((CH,), jnp.int32),
            pltpu.VMEM((CH,), jnp.int32),
            pltpu.VMEM((CH,), jnp.int32),
            pltpu.VMEM((CH,), jnp.int32),
            pltpu.VMEM((CH, D), jnp.float32),
            pltpu.VMEM((SEGS_PER_TILE, D), jnp.float32),
            pltpu.VMEM((SEGS_PER_TILE, D), jnp.float32),
            pltpu.VMEM((REM, D), jnp.float32),
            pltpu.VMEM((REM,), jnp.int32),
            pltpu.VMEM_SHARED((S, D), jnp.float32),
            pltpu.VMEM_SHARED((S, D), jnp.float32),
            pltpu.SemaphoreType.DMA,
            pltpu.SemaphoreType.DMA,
            pltpu.SemaphoreType.DMA,
            pltpu.SemaphoreType.DMA,
            pltpu.SemaphoreType.DMA,
            pltpu.SemaphoreType.DMA,
            pltpu.SemaphoreType.DMA,
            pltpu.SemaphoreType.DMA,
        ],
    )
    psum, pcnt = f(feat, ids32)
    combine = pl.pallas_call(
        _combine_body,
        out_shape=jax.ShapeDtypeStruct((S, D), jnp.float32),
    )
    return combine(psum, pcnt)


def kernel(feat, segment_ids):
    return _segment_mean(feat, segment_ids.astype(jnp.int32))
